# Initial kernel scaffold; baseline (speedup 1.0000x reference)
import jax
import jax.numpy as jnp
from jax import lax
from jax.experimental import pallas as pl
from jax.experimental.pallas import tpu as pltpu
from jax.experimental.pallas import tpu_sc as plsc

LANES = 16
WINDOW = 128


def _embed_kernel(num_indices, embed):
    mesh = plsc.VectorSubcoreMesh(core_axis_name="core", subcore_axis_name="subcore")

    @jax.jit
    def run(idx_flat, weights, aux):
        @pl.kernel(
            out_type=jax.ShapeDtypeStruct((num_indices, embed), jnp.float32),
            mesh=mesh,
        )
        def kernel_body(w_hbm, i_hbm, aux_hbm, o_hbm):
            def body(i_vmem, aux_vmem, o_vmem):
                pltpu.sync_copy(w_hbm.at[i_vmem.at[0]], o_vmem)

                @pl.loop(0, WINDOW, step=LANES)
                def _(j):
                    v = i_vmem[0, pl.ds(j, LANES)]
                    m = v == jnp.int32(1)

                    @pl.when(jnp.any(m))
                    def _():
                        rows = j + lax.broadcasted_iota(jnp.int32, (LANES,), 0)
                        for c in range(embed):
                            plsc.addupdate_scatter(
                                o_vmem,
                                [rows, jnp.full((LANES,), c, jnp.int32)],
                                jnp.full((LANES,), aux_vmem[1, c], jnp.float32),
                                mask=m,
                            )

            pltpu.emit_pipeline(
                body,
                grid=(num_indices // WINDOW,),
                in_specs=[
                    pl.BlockSpec((1, WINDOW), index_map=lambda i: (0, i)),
                    pl.BlockSpec((2, embed), index_map=lambda i: (0, 0)),
                ],
                out_specs=[
                    pl.BlockSpec((WINDOW, embed), index_map=lambda i: (i, 0)),
                ],
                core_axis_name=("core", "subcore"),
                dimension_semantics=(pltpu.PARALLEL,),
            )(i_hbm, aux_hbm, o_hbm)

        return kernel_body(weights, idx_flat, aux)

    return run


def kernel(input_, weights, aux):
    b, l = input_.shape
    num_indices = b * l
    idx_flat = jnp.asarray(input_, jnp.int32).reshape(1, num_indices)
    out = _embed_kernel(num_indices, weights.shape[1])(idx_flat, weights, aux)
    return out.reshape(b, l, weights.shape[1])


# speedup vs baseline: 2.5981x; 2.5981x over previous
import dataclasses

import jax
import jax.numpy as jnp
from jax import lax
from jax.experimental import pallas as pl
from jax.experimental.pallas import tpu as pltpu
from jax.experimental.pallas import tpu_sc as plsc

LANES = 16
WINDOW = 128


def _compiler_params():
    cp = pltpu.CompilerParams(use_tc_tiling_on_sc=False)
    if "needs_layout_passes" in pltpu.CompilerParams.__dataclass_fields__:
        cp = dataclasses.replace(cp, needs_layout_passes=False)
    return cp


def _embed_kernel(num_indices, embed):
    mesh = plsc.VectorSubcoreMesh(core_axis_name="core", subcore_axis_name="subcore")

    @jax.jit
    def run(idx_flat, weights, aux):
        @pl.kernel(
            out_type=jax.ShapeDtypeStruct((num_indices, embed), jnp.float32),
            mesh=mesh,
            compiler_params=_compiler_params(),
        )
        def kernel_body(w_hbm, i_hbm, aux_hbm, o_hbm):
            def body(i_vmem, aux_vmem, o_vmem):
                pltpu.sync_copy(w_hbm.at[i_vmem.at[0]], o_vmem)

                @pl.loop(0, WINDOW, step=LANES)
                def _(j):
                    v = i_vmem[0, pl.ds(j, LANES)]
                    m = v == jnp.int32(1)

                    @pl.when(jnp.any(m))
                    def _():
                        rows = j + lax.broadcasted_iota(jnp.int32, (LANES,), 0)
                        for cb in range(embed // LANES):
                            av = aux_vmem[1, pl.ds(cb * LANES, LANES)]
                            for t in range(LANES):
                                c = cb * LANES + t
                                plsc.addupdate_scatter(
                                    o_vmem,
                                    [rows, jnp.full((LANES,), c, jnp.int32)],
                                    jnp.full((LANES,), av[t], jnp.float32),
                                    mask=m,
                                )

            pltpu.emit_pipeline(
                body,
                grid=(num_indices // WINDOW,),
                in_specs=[
                    pl.BlockSpec((1, WINDOW), index_map=lambda i: (0, i)),
                    pl.BlockSpec((2, embed), index_map=lambda i: (0, 0)),
                ],
                out_specs=[
                    pl.BlockSpec((WINDOW, embed), index_map=lambda i: (i, 0)),
                ],
                core_axis_name=("core", "subcore"),
                dimension_semantics=(pltpu.PARALLEL,),
            )(i_hbm, aux_hbm, o_hbm)

        return kernel_body(weights, idx_flat, aux)

    return run


def kernel(input_, weights, aux):
    b, l = input_.shape
    num_indices = b * l
    idx_flat = jnp.asarray(input_, jnp.int32).reshape(1, num_indices)
    out = _embed_kernel(num_indices, weights.shape[1])(idx_flat, weights, aux)
    return out.reshape(b, l, weights.shape[1])
